# Initial kernel scaffold; baseline (speedup 1.0000x reference)
import jax
import jax.numpy as jnp
from jax.experimental import pallas as pl


def kernel(x, node_emb, time_in_day_emb, day_in_week_emb, W, b):
    raise NotImplementedError("write your pallas kernel here")


# speedup vs baseline: 5.2486x; 5.2486x over previous
import jax
import jax.numpy as jnp
from jax.experimental import pallas as pl

B, L, N, C = 8, 12, 50000, 3
EDIM = 32
LC = L * C

BLOCK_N = 2560
NUM_NB = -(-N // BLOCK_N)


def _stid_kernel(xt_ref, w_ref, b_ref, node_ref, tidT_ref, diwT_ref, out_ref):
    xt = xt_ref[0]
    ts = jax.lax.dot_general(
        w_ref[...], xt, (((1,), (0,)), ((), ())),
        preferred_element_type=jnp.float32)
    ts = ts + b_ref[...]

    tid_f = xt[34:35, :]
    diw_f = xt[35:36, :]
    tid_idx = jnp.clip((tid_f * 288.0).astype(jnp.int32), 0, 287)
    diw_idx = jnp.clip((diw_f * 7.0).astype(jnp.int32), 0, 6)

    iota288 = jax.lax.broadcasted_iota(jnp.int32, (288, BLOCK_N), 0)
    onehot_tid = (iota288 == tid_idx).astype(jnp.float32)
    tid = jax.lax.dot_general(
        tidT_ref[...], onehot_tid, (((1,), (0,)), ((), ())),
        preferred_element_type=jnp.float32)

    iota7 = jax.lax.broadcasted_iota(jnp.int32, (7, BLOCK_N), 0)
    onehot_diw = (iota7 == diw_idx).astype(jnp.float32)
    diw = jax.lax.dot_general(
        diwT_ref[...], onehot_diw, (((1,), (0,)), ((), ())),
        preferred_element_type=jnp.float32)

    out_ref[0, 0:32, :] = ts
    out_ref[0, 32:64, :] = node_ref[...]
    out_ref[0, 64:96, :] = tid
    out_ref[0, 96:128, :] = diw


def kernel(x, node_emb, time_in_day_emb, day_in_week_emb, W, b):
    xt = jnp.transpose(x, (0, 1, 3, 2)).reshape(B, LC, N)
    nodeT = node_emb.T
    tidT = time_in_day_emb.T
    diwT = day_in_week_emb.T
    b2 = b.reshape(EDIM, 1)

    out = pl.pallas_call(
        _stid_kernel,
        grid=(NUM_NB, B),
        in_specs=[
            pl.BlockSpec((1, LC, BLOCK_N), lambda nb, bb: (bb, 0, nb)),
            pl.BlockSpec((EDIM, LC), lambda nb, bb: (0, 0)),
            pl.BlockSpec((EDIM, 1), lambda nb, bb: (0, 0)),
            pl.BlockSpec((EDIM, BLOCK_N), lambda nb, bb: (0, nb)),
            pl.BlockSpec((EDIM, 288), lambda nb, bb: (0, 0)),
            pl.BlockSpec((EDIM, 7), lambda nb, bb: (0, 0)),
        ],
        out_specs=pl.BlockSpec((1, 4 * EDIM, BLOCK_N), lambda nb, bb: (bb, 0, nb)),
        out_shape=jax.ShapeDtypeStruct((B, 4 * EDIM, N), jnp.float32),
    )(xt, W, b2, nodeT, tidT, diwT)
    return out[..., None]
